# baseline (speedup 1.0000x reference)
import functools

import jax
import jax.numpy as jnp
from jax import lax
from jax.experimental import pallas as pl
from jax.experimental.pallas import tpu as pltpu
from jax.experimental.pallas import tpu_sc as plsc

BATCH = 16384
EMBED = 32
NC = 2
NS = 16
NW = NC * NS
B_PER_W = BATCH // NW
CHUNK = 128
NCHUNK = B_PER_W // CHUNK
GROUPS = B_PER_W // 16


def _body(uidx_hbm, iidx_hbm, utab_hbm, itab_hbm, out_hbm,
          uidx_v, iidx_v, urows_v, irows_v, out_v, sem):
  wid = lax.axis_index("s") * NC + lax.axis_index("c")
  base = wid * B_PER_W

  pltpu.sync_copy(uidx_hbm.at[pl.ds(wid * NCHUNK, NCHUNK)], uidx_v)
  pltpu.sync_copy(iidx_hbm.at[pl.ds(wid * NCHUNK, NCHUNK)], iidx_v)

  copies = []
  for j in range(NCHUNK):
    copies.append(pltpu.async_copy(
        utab_hbm.at[uidx_v.at[j]], urows_v.at[pl.ds(j * CHUNK, CHUNK)], sem))
    copies.append(pltpu.async_copy(
        itab_hbm.at[iidx_v.at[j]], irows_v.at[pl.ds(j * CHUNK, CHUNK)], sem))
  for c in copies:
    c.wait()

  lane = lax.iota(jnp.int32, 16)

  def group(g, carry):
    r0 = g * 16
    row_idx = r0 + lane
    acc = jnp.zeros((16,), jnp.float32)
    for d in range(EMBED):
      col = jnp.full((16,), d, jnp.int32)
      cu = plsc.load_gather(urows_v, [row_idx, col])
      ci = plsc.load_gather(irows_v, [row_idx, col])
      acc = acc + cu * ci
    out_v[pl.ds(r0, 16)] = acc
    return carry

  lax.fori_loop(0, GROUPS, group, 0)

  pltpu.sync_copy(out_v, out_hbm.at[pl.ds(base, B_PER_W)])


@functools.partial(
    pl.kernel,
    out_type=jax.ShapeDtypeStruct((BATCH,), jnp.float32),
    mesh=plsc.VectorSubcoreMesh(core_axis_name="c", subcore_axis_name="s",
                                num_cores=NC, num_subcores=NS),
    compiler_params=pltpu.CompilerParams(needs_layout_passes=False,
                                         use_tc_tiling_on_sc=False),
    scratch_types=[
        pltpu.VMEM((NCHUNK, CHUNK), jnp.int32),
        pltpu.VMEM((NCHUNK, CHUNK), jnp.int32),
        pltpu.VMEM((B_PER_W, EMBED), jnp.float32),
        pltpu.VMEM((B_PER_W, EMBED), jnp.float32),
        pltpu.VMEM((B_PER_W,), jnp.float32),
        pltpu.SemaphoreType.DMA,
    ],
)
def _sc_dot(uidx_hbm, iidx_hbm, utab_hbm, itab_hbm, out_hbm,
            uidx_v, iidx_v, urows_v, irows_v, out_v, sem):
  _body(uidx_hbm, iidx_hbm, utab_hbm, itab_hbm, out_hbm,
        uidx_v, iidx_v, urows_v, irows_v, out_v, sem)


def kernel(inputs, user_table, item_table):
  uidx = inputs[:, 0].reshape(NW * NCHUNK, CHUNK)
  iidx = inputs[:, 1].reshape(NW * NCHUNK, CHUNK)
  return _sc_dot(uidx, iidx, user_table, item_table)
